# baseline (speedup 1.0000x reference)
import functools

import jax
import jax.numpy as jnp
from jax import lax
from jax.experimental import pallas as pl
from jax.experimental.pallas import tpu as pltpu
from jax.experimental.pallas import tpu_sc as plsc

N = 10000
E = 320000
DIN = 128
DH = 64
TW = 128
G = 64
NCLS = 10

NC, NS = 2, 16
NW = NC * NS
CHUNK = 128
CPW = 80
E_PAD = NW * CPW * CHUNK
ACC_ROWS = N + 8
RPT = 632
RPT_LAST = N - (NS - 1) * RPT

NBLK = 10
BLK = N // NBLK


@functools.cache
def _make_sc_pass():
    mesh = plsc.VectorSubcoreMesh(core_axis_name="c", subcore_axis_name="s",
                                  num_cores=NC, num_subcores=NS)

    def body(src_hbm, dst_hbm, p_hbm, z_hbm, agg_out,
             acc_sh, srcv, dstv, rowsv, sem):
        c = lax.axis_index("c")
        s = lax.axis_index("s")
        wid = s * NC + c
        row0 = pl.multiple_of(s * RPT, 8)

        def on_slice(fn):
            @pl.when(s < NS - 1)
            def _():
                fn(row0, RPT)

            @pl.when(s == NS - 1)
            def _():
                fn(row0, RPT_LAST)

        on_slice(lambda r0, nr: pltpu.sync_copy(z_hbm.at[pl.ds(r0, nr)],
                                                acc_sh.at[pl.ds(r0, nr)]))
        plsc.subcore_barrier()

        def chunk_body(i, carry):
            base = pl.multiple_of((wid + NW * i) * CHUNK, 8)
            pltpu.sync_copy(src_hbm.at[pl.ds(base, CHUNK)], srcv)
            pltpu.sync_copy(dst_hbm.at[pl.ds(base, CHUNK)], dstv)
            pltpu.async_copy(p_hbm.at[srcv], rowsv, sem).wait()
            pltpu.sync_copy(rowsv, acc_sh.at[dstv], add=True)
            return carry

        lax.fori_loop(0, CPW, chunk_body, 0)
        plsc.subcore_barrier()
        on_slice(lambda r0, nr: pltpu.sync_copy(
            acc_sh.at[pl.ds(r0, nr)], agg_out.at[c, pl.ds(r0, nr)]))

    return pl.kernel(
        body,
        out_type=jax.ShapeDtypeStruct((NC, N, TW), jnp.float32),
        mesh=mesh,
        scratch_types=[
            pltpu.VMEM_SHARED((ACC_ROWS, TW), jnp.float32),
            pltpu.VMEM((CHUNK,), jnp.int32),
            pltpu.VMEM((CHUNK,), jnp.int32),
            pltpu.VMEM((CHUNK, TW), jnp.float32),
            pltpu.SemaphoreType.DMA,
        ])


def _write_table(p_ref, proj):
    p_ref[:, 0:DH] = proj
    p_ref[:, DH:] = jnp.concatenate(
        [jnp.ones((proj.shape[0], 1), jnp.float32),
         jnp.zeros((proj.shape[0], TW - DH - 1), jnp.float32)], axis=1)


def _dense_in_body(x_ref, wl_ref, wr_ref, b_ref, p_ref, r_ref):
    xb = x_ref[...]
    _write_table(p_ref,
                 jnp.dot(xb, wl_ref[...], preferred_element_type=jnp.float32))
    r_ref[...] = (jnp.dot(xb, wr_ref[...], preferred_element_type=jnp.float32)
                  + b_ref[...])


def _dense_in(x, wl, wr, b):
    return pl.pallas_call(
        _dense_in_body,
        grid=(NBLK,),
        in_specs=[
            pl.BlockSpec((BLK, DIN), lambda i: (i, 0)),
            pl.BlockSpec((DIN, DH), lambda i: (0, 0)),
            pl.BlockSpec((DIN, DH), lambda i: (0, 0)),
            pl.BlockSpec((1, DH), lambda i: (0, 0)),
        ],
        out_specs=[
            pl.BlockSpec((BLK, TW), lambda i: (i, 0)),
            pl.BlockSpec((BLK, DH), lambda i: (i, 0)),
        ],
        out_shape=[
            jax.ShapeDtypeStruct((N, TW), jnp.float32),
            jax.ShapeDtypeStruct((N, DH), jnp.float32),
        ],
    )(x, wl, wr, b.reshape(1, DH))


def _node_update(agg_ref, deg_ref, r_ref):
    agg = agg_ref[0, :, 0:DH] + agg_ref[1, :, 0:DH]
    deg = deg_ref[0, :, DH:DH + 1] + deg_ref[1, :, DH:DH + 1]
    return agg / jnp.maximum(deg, 1.0) + r_ref[...]


def _dense_mid_body(agg_ref, deg_ref, r_ref, wl_ref, wr_ref, b_ref,
                    p_ref, rn_ref):
    xi = _node_update(agg_ref, deg_ref, r_ref)
    _write_table(p_ref,
                 jnp.dot(xi, wl_ref[...], preferred_element_type=jnp.float32))
    rn_ref[...] = (jnp.dot(xi, wr_ref[...], preferred_element_type=jnp.float32)
                   + b_ref[...])


def _dense_mid(agg, deg_carrier, r_prev, wl, wr, b):
    return pl.pallas_call(
        _dense_mid_body,
        grid=(NBLK,),
        in_specs=[
            pl.BlockSpec((NC, BLK, TW), lambda i: (0, i, 0)),
            pl.BlockSpec((NC, BLK, TW), lambda i: (0, i, 0)),
            pl.BlockSpec((BLK, DH), lambda i: (i, 0)),
            pl.BlockSpec((DH, DH), lambda i: (0, 0)),
            pl.BlockSpec((DH, DH), lambda i: (0, 0)),
            pl.BlockSpec((1, DH), lambda i: (0, 0)),
        ],
        out_specs=[
            pl.BlockSpec((BLK, TW), lambda i: (i, 0)),
            pl.BlockSpec((BLK, DH), lambda i: (i, 0)),
        ],
        out_shape=[
            jax.ShapeDtypeStruct((N, TW), jnp.float32),
            jax.ShapeDtypeStruct((N, DH), jnp.float32),
        ],
    )(agg, deg_carrier, r_prev, wl, wr, b.reshape(1, DH))


def _final_body(agg_ref, deg_ref, r_ref, batch_ref,
                w1_ref, c1_ref, w2_ref, c2_ref, w3_ref, c3_ref,
                w4_ref, b4_ref, out_ref, pool_acc, cnt_acc):
    i = pl.program_id(0)

    @pl.when(i == 0)
    def _():
        pool_acc[...] = jnp.zeros_like(pool_acc)
        cnt_acc[...] = jnp.zeros_like(cnt_acc)

    xi = _node_update(agg_ref, deg_ref, r_ref)
    gids = lax.broadcasted_iota(jnp.int32, (BLK, G), 1)
    oh = (batch_ref[...] == gids).astype(jnp.float32)
    pool_acc[...] += lax.dot_general(
        oh, xi, (((0,), (0,)), ((), ())),
        preferred_element_type=jnp.float32)
    cnt_acc[...] += lax.dot_general(
        oh, jnp.ones((BLK, 1), jnp.float32), (((0,), (0,)), ((), ())),
        preferred_element_type=jnp.float32)

    @pl.when(i == NBLK - 1)
    def _():
        z = pool_acc[...] / jnp.maximum(cnt_acc[...], 1.0)
        z = jnp.tanh(jnp.dot(z, w1_ref[...],
                             preferred_element_type=jnp.float32) + c1_ref[...])
        z = jnp.tanh(jnp.dot(z, w2_ref[...],
                             preferred_element_type=jnp.float32) + c2_ref[...])
        z = jnp.tanh(jnp.dot(z, w3_ref[...],
                             preferred_element_type=jnp.float32) + c3_ref[...])
        out_ref[...] = (jnp.dot(z, w4_ref[...],
                                preferred_element_type=jnp.float32)
                        + b4_ref[...])


def _final(agg, deg_carrier, r3, batch_col, w1, c1, w2, c2, w3, c3, w4, b4):
    full = lambda a: pl.BlockSpec(a.shape, lambda i: tuple(0 for _ in a.shape))
    return pl.pallas_call(
        _final_body,
        grid=(NBLK,),
        in_specs=[
            pl.BlockSpec((NC, BLK, TW), lambda i: (0, i, 0)),
            pl.BlockSpec((NC, BLK, TW), lambda i: (0, i, 0)),
            pl.BlockSpec((BLK, DH), lambda i: (i, 0)),
            pl.BlockSpec((BLK, 1), lambda i: (i, 0)),
            full(w1), full(c1), full(w2), full(c2),
            full(w3), full(c3), full(w4), full(b4),
        ],
        out_specs=pl.BlockSpec((G, NCLS), lambda i: (0, 0)),
        out_shape=jax.ShapeDtypeStruct((G, NCLS), jnp.float32),
        scratch_shapes=[
            pltpu.VMEM((G, DH), jnp.float32),
            pltpu.VMEM((G, 1), jnp.float32),
        ],
    )(agg, deg_carrier, r3, batch_col, w1, c1, w2, c2, w3, c3, w4, b4)


def kernel(x, edge_index, batch,
           cW1l, cW1r, cb1, cW2l, cW2r, cb2, cW3l, cW3r, cb3,
           W1, b1, g1, be1, rm1, rv1,
           W2, b2, g2, be2, rm2, rv2,
           W3, b3, g3, be3, rm3, rv3,
           W4, b4):
    npad = E_PAD - E
    src = jnp.concatenate([edge_index[0], jnp.zeros((npad,), jnp.int32)])
    dst = jnp.concatenate([edge_index[1], jnp.full((npad,), N, jnp.int32)])
    z = jnp.zeros((N, TW), jnp.float32)

    def fold(w, b, gm, be, rm, rv):
        s = gm * jax.lax.rsqrt(rv + 1e-5)
        return w * s[None, :], ((b - rm) * s + be).reshape(1, -1)

    w1f, c1 = fold(W1, b1, g1, be1, rm1, rv1)
    w2f, c2 = fold(W2, b2, g2, be2, rm2, rv2)
    w3f, c3 = fold(W3, b3, g3, be3, rm3, rv3)

    sc_pass = _make_sc_pass()
    p1, r1 = _dense_in(x, cW1l, cW1r, cb1)
    agg1 = sc_pass(src, dst, p1, z)
    p2, r2 = _dense_mid(agg1, agg1, r1, cW2l, cW2r, cb2)
    agg2 = sc_pass(src, dst, p2, z)
    p3, r3 = _dense_mid(agg2, agg1, r2, cW3l, cW3r, cb3)
    agg3 = sc_pass(src, dst, p3, z)
    return _final(agg3, agg1, r3, batch.reshape(N, 1),
                  w1f, c1, w2f, c2, w3f, c3, W4, b4.reshape(1, NCLS))


# speedup vs baseline: 1.9988x; 1.9988x over previous
import functools

import jax
import jax.numpy as jnp
from jax import lax
from jax.experimental import pallas as pl
from jax.experimental.pallas import tpu as pltpu
from jax.experimental.pallas import tpu_sc as plsc

N = 10000
E = 320000
DIN = 128
DH = 64
TW = 128
G = 64
NCLS = 10

NC, NS = 2, 16
NW = NC * NS
CHUNK = 128
NCHUNKS = E // CHUNK
CPW = -(-NCHUNKS // NW)
RPT = 632
RPT_LAST = N - (NS - 1) * RPT

NBLK = 10
BLK = N // NBLK


@functools.cache
def _make_sc_pass():
    mesh = plsc.VectorSubcoreMesh(core_axis_name="c", subcore_axis_name="s",
                                  num_cores=NC, num_subcores=NS)

    def body(src_hbm, dst_hbm, p_hbm, z_hbm, agg_out,
             acc_sh, srcv, dstv, rowsv, sem):
        c = lax.axis_index("c")
        s = lax.axis_index("s")
        wid = s * NC + c
        row0 = pl.multiple_of(s * RPT, 8)

        def on_slice(fn):
            @pl.when(s < NS - 1)
            def _():
                fn(row0, RPT)

            @pl.when(s == NS - 1)
            def _():
                fn(row0, RPT_LAST)

        on_slice(lambda r0, nr: pltpu.sync_copy(z_hbm.at[pl.ds(r0, nr)],
                                                acc_sh.at[pl.ds(r0, nr)]))
        plsc.subcore_barrier()

        def chunk_body(i, carry):
            chunk_id = wid + NW * i

            @pl.when(chunk_id < NCHUNKS)
            def _():
                base = chunk_id * CHUNK
                pltpu.sync_copy(src_hbm.at[pl.ds(base, CHUNK)], srcv)
                pltpu.sync_copy(dst_hbm.at[pl.ds(base, CHUNK)], dstv)
                pltpu.async_copy(p_hbm.at[srcv], rowsv, sem).wait()
                pltpu.sync_copy(rowsv, acc_sh.at[dstv], add=True)
            return carry

        lax.fori_loop(0, CPW, chunk_body, 0)
        plsc.subcore_barrier()
        on_slice(lambda r0, nr: pltpu.sync_copy(
            acc_sh.at[pl.ds(r0, nr)], agg_out.at[c, pl.ds(r0, nr)]))

    return pl.kernel(
        body,
        out_type=jax.ShapeDtypeStruct((NC, N, TW), jnp.float32),
        mesh=mesh,
        scratch_types=[
            pltpu.VMEM_SHARED((N, TW), jnp.float32),
            pltpu.VMEM((CHUNK,), jnp.int32),
            pltpu.VMEM((CHUNK,), jnp.int32),
            pltpu.VMEM((CHUNK, TW), jnp.float32),
            pltpu.SemaphoreType.DMA,
        ])


def _write_table(p_ref, proj):
    p_ref[:, 0:DH] = proj
    p_ref[:, DH:] = jnp.concatenate(
        [jnp.ones((proj.shape[0], 1), jnp.float32),
         jnp.zeros((proj.shape[0], TW - DH - 1), jnp.float32)], axis=1)


def _dense_in_body(x_ref, wl_ref, wr_ref, b_ref, p_ref, r_ref):
    xb = x_ref[...]
    _write_table(p_ref,
                 jnp.dot(xb, wl_ref[...], preferred_element_type=jnp.float32))
    r_ref[...] = (jnp.dot(xb, wr_ref[...], preferred_element_type=jnp.float32)
                  + b_ref[...])


def _dense_in(x, wl, wr, b):
    return pl.pallas_call(
        _dense_in_body,
        grid=(NBLK,),
        in_specs=[
            pl.BlockSpec((BLK, DIN), lambda i: (i, 0)),
            pl.BlockSpec((DIN, DH), lambda i: (0, 0)),
            pl.BlockSpec((DIN, DH), lambda i: (0, 0)),
            pl.BlockSpec((1, DH), lambda i: (0, 0)),
        ],
        out_specs=[
            pl.BlockSpec((BLK, TW), lambda i: (i, 0)),
            pl.BlockSpec((BLK, DH), lambda i: (i, 0)),
        ],
        out_shape=[
            jax.ShapeDtypeStruct((N, TW), jnp.float32),
            jax.ShapeDtypeStruct((N, DH), jnp.float32),
        ],
    )(x, wl, wr, b.reshape(1, DH))


def _node_update(agg_ref, deg_ref, r_ref):
    agg = agg_ref[0, :, 0:DH] + agg_ref[1, :, 0:DH]
    deg = deg_ref[0, :, DH:DH + 1] + deg_ref[1, :, DH:DH + 1]
    return agg / jnp.maximum(deg, 1.0) + r_ref[...]


def _dense_mid_body(agg_ref, deg_ref, r_ref, wl_ref, wr_ref, b_ref,
                    p_ref, rn_ref):
    xi = _node_update(agg_ref, deg_ref, r_ref)
    _write_table(p_ref,
                 jnp.dot(xi, wl_ref[...], preferred_element_type=jnp.float32))
    rn_ref[...] = (jnp.dot(xi, wr_ref[...], preferred_element_type=jnp.float32)
                   + b_ref[...])


def _dense_mid(agg, deg_carrier, r_prev, wl, wr, b):
    return pl.pallas_call(
        _dense_mid_body,
        grid=(NBLK,),
        in_specs=[
            pl.BlockSpec((NC, BLK, TW), lambda i: (0, i, 0)),
            pl.BlockSpec((NC, BLK, TW), lambda i: (0, i, 0)),
            pl.BlockSpec((BLK, DH), lambda i: (i, 0)),
            pl.BlockSpec((DH, DH), lambda i: (0, 0)),
            pl.BlockSpec((DH, DH), lambda i: (0, 0)),
            pl.BlockSpec((1, DH), lambda i: (0, 0)),
        ],
        out_specs=[
            pl.BlockSpec((BLK, TW), lambda i: (i, 0)),
            pl.BlockSpec((BLK, DH), lambda i: (i, 0)),
        ],
        out_shape=[
            jax.ShapeDtypeStruct((N, TW), jnp.float32),
            jax.ShapeDtypeStruct((N, DH), jnp.float32),
        ],
    )(agg, deg_carrier, r_prev, wl, wr, b.reshape(1, DH))


def _final_body(agg_ref, deg_ref, r_ref, batch_ref,
                w1_ref, c1_ref, w2_ref, c2_ref, w3_ref, c3_ref,
                w4_ref, b4_ref, out_ref, pool_acc, cnt_acc):
    i = pl.program_id(0)

    @pl.when(i == 0)
    def _():
        pool_acc[...] = jnp.zeros_like(pool_acc)
        cnt_acc[...] = jnp.zeros_like(cnt_acc)

    xi = _node_update(agg_ref, deg_ref, r_ref)
    gids = lax.broadcasted_iota(jnp.int32, (BLK, G), 1)
    oh = (batch_ref[...] == gids).astype(jnp.float32)
    pool_acc[...] += lax.dot_general(
        oh, xi, (((0,), (0,)), ((), ())),
        preferred_element_type=jnp.float32)
    cnt_acc[...] += lax.dot_general(
        oh, jnp.ones((BLK, 1), jnp.float32), (((0,), (0,)), ((), ())),
        preferred_element_type=jnp.float32)

    @pl.when(i == NBLK - 1)
    def _():
        z = pool_acc[...] / jnp.maximum(cnt_acc[...], 1.0)
        z = jnp.tanh(jnp.dot(z, w1_ref[...],
                             preferred_element_type=jnp.float32) + c1_ref[...])
        z = jnp.tanh(jnp.dot(z, w2_ref[...],
                             preferred_element_type=jnp.float32) + c2_ref[...])
        z = jnp.tanh(jnp.dot(z, w3_ref[...],
                             preferred_element_type=jnp.float32) + c3_ref[...])
        out_ref[...] = (jnp.dot(z, w4_ref[...],
                                preferred_element_type=jnp.float32)
                        + b4_ref[...])


def _final(agg, deg_carrier, r3, batch_col, w1, c1, w2, c2, w3, c3, w4, b4):
    full = lambda a: pl.BlockSpec(a.shape, lambda i: tuple(0 for _ in a.shape))
    return pl.pallas_call(
        _final_body,
        grid=(NBLK,),
        in_specs=[
            pl.BlockSpec((NC, BLK, TW), lambda i: (0, i, 0)),
            pl.BlockSpec((NC, BLK, TW), lambda i: (0, i, 0)),
            pl.BlockSpec((BLK, DH), lambda i: (i, 0)),
            pl.BlockSpec((BLK, 1), lambda i: (i, 0)),
            full(w1), full(c1), full(w2), full(c2),
            full(w3), full(c3), full(w4), full(b4),
        ],
        out_specs=pl.BlockSpec((G, NCLS), lambda i: (0, 0)),
        out_shape=jax.ShapeDtypeStruct((G, NCLS), jnp.float32),
        scratch_shapes=[
            pltpu.VMEM((G, DH), jnp.float32),
            pltpu.VMEM((G, 1), jnp.float32),
        ],
    )(agg, deg_carrier, r3, batch_col, w1, c1, w2, c2, w3, c3, w4, b4)


def kernel(x, edge_index, batch,
           cW1l, cW1r, cb1, cW2l, cW2r, cb2, cW3l, cW3r, cb3,
           W1, b1, g1, be1, rm1, rv1,
           W2, b2, g2, be2, rm2, rv2,
           W3, b3, g3, be3, rm3, rv3,
           W4, b4):
    src = edge_index[0]
    dst = edge_index[1]
    z = jnp.zeros((N, TW), jnp.float32)

    def fold(w, b, gm, be, rm, rv):
        s = gm * jax.lax.rsqrt(rv + 1e-5)
        return w * s[None, :], ((b - rm) * s + be).reshape(1, -1)

    w1f, c1 = fold(W1, b1, g1, be1, rm1, rv1)
    w2f, c2 = fold(W2, b2, g2, be2, rm2, rv2)
    w3f, c3 = fold(W3, b3, g3, be3, rm3, rv3)

    sc_pass = _make_sc_pass()
    p1, r1 = _dense_in(x, cW1l, cW1r, cb1)
    agg1 = sc_pass(src, dst, p1, z)
    p2, r2 = _dense_mid(agg1, agg1, r1, cW2l, cW2r, cb2)
    agg2 = sc_pass(src, dst, p2, z)
    p3, r3 = _dense_mid(agg2, agg1, r2, cW3l, cW3r, cb3)
    agg3 = sc_pass(src, dst, p3, z)
    return _final(agg3, agg1, r3, batch.reshape(N, 1),
                  w1f, c1, w2f, c2, w3f, c3, W4, b4.reshape(1, NCLS))


# speedup vs baseline: 3.0680x; 1.5350x over previous
import functools

import jax
import jax.numpy as jnp
from jax import lax
from jax.experimental import pallas as pl
from jax.experimental.pallas import tpu as pltpu
from jax.experimental.pallas import tpu_sc as plsc

N = 10000
E = 320000
DIN = 128
DH = 64
TW = 128
G = 64
NCLS = 10

NC, NS = 2, 16
NW = NC * NS
CHUNK = 128
NCHUNKS = E // CHUNK
CPW = -(-NCHUNKS // NW)
RPT = 632
RPT_LAST = N - (NS - 1) * RPT

NBLK = 10
BLK = N // NBLK


@functools.cache
def _make_sc_pass():
    mesh = plsc.VectorSubcoreMesh(core_axis_name="c", subcore_axis_name="s",
                                  num_cores=NC, num_subcores=NS)

    def body(src_hbm, dst_hbm, p_hbm, z_hbm, agg_out,
             acc_sh, src0, dst0, src1, dst1, rows0, rows1, sem0, sem1):
        c = lax.axis_index("c")
        s = lax.axis_index("s")
        wid = s * NC + c
        row0 = pl.multiple_of(s * RPT, 8)
        srcb = (src0, src1)
        dstb = (dst0, dst1)
        rowsb = (rows0, rows1)
        semb = (sem0, sem1)
        CH = NCHUNKS // NW
        cbase = wid * CH

        def on_slice(fn):
            @pl.when(s < NS - 1)
            def _():
                fn(row0, RPT)

            @pl.when(s == NS - 1)
            def _():
                fn(row0, RPT_LAST)

        on_slice(lambda r0, nr: pltpu.sync_copy(z_hbm.at[pl.ds(r0, nr)],
                                                acc_sh.at[pl.ds(r0, nr)]))
        plsc.subcore_barrier()

        def idx_load(j, b):
            base = pl.multiple_of((cbase + j) * CHUNK, 8)
            pltpu.sync_copy(src_hbm.at[pl.ds(base, CHUNK)], srcb[b])
            pltpu.sync_copy(dst_hbm.at[pl.ds(base, CHUNK)], dstb[b])

        def gfire(b):
            pltpu.async_copy(p_hbm.at[srcb[b]], rowsb[b], semb[b])

        def gwait(b):
            pltpu.make_async_copy(p_hbm.at[srcb[b]], rowsb[b], semb[b]).wait()

        def scat(b):
            pltpu.sync_copy(rowsb[b], acc_sh.at[dstb[b]], add=True)

        idx_load(0, 0)
        gfire(0)

        def chunk_body(i, carry):
            idx_load(2 * i + 1, 1)
            gfire(1)
            gwait(0)
            scat(0)
            idx_load(2 * i + 2, 0)
            gfire(0)
            gwait(1)
            scat(1)
            return carry

        lax.fori_loop(0, CH // 2 - 1, chunk_body, 0)
        idx_load(CH - 1, 1)
        gfire(1)
        gwait(0)
        scat(0)
        gwait(1)
        scat(1)

        @pl.when(wid < 4)
        def _():
            base = pl.multiple_of((NW * CH + wid) * CHUNK, 8)
            pltpu.sync_copy(src_hbm.at[pl.ds(base, CHUNK)], src0)
            pltpu.sync_copy(dst_hbm.at[pl.ds(base, CHUNK)], dst0)
            pltpu.async_copy(p_hbm.at[src0], rows0, sem0).wait()
            pltpu.sync_copy(rows0, acc_sh.at[dst0], add=True)

        plsc.subcore_barrier()
        on_slice(lambda r0, nr: pltpu.sync_copy(
            acc_sh.at[pl.ds(r0, nr)], agg_out.at[c, pl.ds(r0, nr)]))

    return pl.kernel(
        body,
        out_type=jax.ShapeDtypeStruct((NC, N, TW), jnp.float32),
        mesh=mesh,
        scratch_types=[
            pltpu.VMEM_SHARED((N, TW), jnp.float32),
            pltpu.VMEM((CHUNK,), jnp.int32),
            pltpu.VMEM((CHUNK,), jnp.int32),
            pltpu.VMEM((CHUNK,), jnp.int32),
            pltpu.VMEM((CHUNK,), jnp.int32),
            pltpu.VMEM((CHUNK, TW), jnp.float32),
            pltpu.VMEM((CHUNK, TW), jnp.float32),
            pltpu.SemaphoreType.DMA,
            pltpu.SemaphoreType.DMA,
        ])


def _write_table(p_ref, proj):
    p_ref[:, 0:DH] = proj
    p_ref[:, DH:] = jnp.concatenate(
        [jnp.ones((proj.shape[0], 1), jnp.float32),
         jnp.zeros((proj.shape[0], TW - DH - 1), jnp.float32)], axis=1)


def _dense_in_body(x_ref, wl_ref, wr_ref, b_ref, p_ref, r_ref):
    xb = x_ref[...]
    _write_table(p_ref,
                 jnp.dot(xb, wl_ref[...], preferred_element_type=jnp.float32))
    r_ref[...] = (jnp.dot(xb, wr_ref[...], preferred_element_type=jnp.float32)
                  + b_ref[...])


def _dense_in(x, wl, wr, b):
    return pl.pallas_call(
        _dense_in_body,
        grid=(NBLK,),
        in_specs=[
            pl.BlockSpec((BLK, DIN), lambda i: (i, 0)),
            pl.BlockSpec((DIN, DH), lambda i: (0, 0)),
            pl.BlockSpec((DIN, DH), lambda i: (0, 0)),
            pl.BlockSpec((1, DH), lambda i: (0, 0)),
        ],
        out_specs=[
            pl.BlockSpec((BLK, TW), lambda i: (i, 0)),
            pl.BlockSpec((BLK, DH), lambda i: (i, 0)),
        ],
        out_shape=[
            jax.ShapeDtypeStruct((N, TW), jnp.float32),
            jax.ShapeDtypeStruct((N, DH), jnp.float32),
        ],
    )(x, wl, wr, b.reshape(1, DH))


def _node_update(agg_ref, deg_ref, r_ref):
    agg = agg_ref[0, :, 0:DH] + agg_ref[1, :, 0:DH]
    deg = deg_ref[0, :, DH:DH + 1] + deg_ref[1, :, DH:DH + 1]
    return agg / jnp.maximum(deg, 1.0) + r_ref[...]


def _dense_mid_body(agg_ref, deg_ref, r_ref, wl_ref, wr_ref, b_ref,
                    p_ref, rn_ref):
    xi = _node_update(agg_ref, deg_ref, r_ref)
    _write_table(p_ref,
                 jnp.dot(xi, wl_ref[...], preferred_element_type=jnp.float32))
    rn_ref[...] = (jnp.dot(xi, wr_ref[...], preferred_element_type=jnp.float32)
                   + b_ref[...])


def _dense_mid(agg, deg_carrier, r_prev, wl, wr, b):
    return pl.pallas_call(
        _dense_mid_body,
        grid=(NBLK,),
        in_specs=[
            pl.BlockSpec((NC, BLK, TW), lambda i: (0, i, 0)),
            pl.BlockSpec((NC, BLK, TW), lambda i: (0, i, 0)),
            pl.BlockSpec((BLK, DH), lambda i: (i, 0)),
            pl.BlockSpec((DH, DH), lambda i: (0, 0)),
            pl.BlockSpec((DH, DH), lambda i: (0, 0)),
            pl.BlockSpec((1, DH), lambda i: (0, 0)),
        ],
        out_specs=[
            pl.BlockSpec((BLK, TW), lambda i: (i, 0)),
            pl.BlockSpec((BLK, DH), lambda i: (i, 0)),
        ],
        out_shape=[
            jax.ShapeDtypeStruct((N, TW), jnp.float32),
            jax.ShapeDtypeStruct((N, DH), jnp.float32),
        ],
    )(agg, deg_carrier, r_prev, wl, wr, b.reshape(1, DH))


def _final_body(agg_ref, deg_ref, r_ref, batch_ref,
                w1_ref, c1_ref, w2_ref, c2_ref, w3_ref, c3_ref,
                w4_ref, b4_ref, out_ref, pool_acc, cnt_acc):
    i = pl.program_id(0)

    @pl.when(i == 0)
    def _():
        pool_acc[...] = jnp.zeros_like(pool_acc)
        cnt_acc[...] = jnp.zeros_like(cnt_acc)

    xi = _node_update(agg_ref, deg_ref, r_ref)
    gids = lax.broadcasted_iota(jnp.int32, (BLK, G), 1)
    oh = (batch_ref[...] == gids).astype(jnp.float32)
    pool_acc[...] += lax.dot_general(
        oh, xi, (((0,), (0,)), ((), ())),
        preferred_element_type=jnp.float32)
    cnt_acc[...] += lax.dot_general(
        oh, jnp.ones((BLK, 1), jnp.float32), (((0,), (0,)), ((), ())),
        preferred_element_type=jnp.float32)

    @pl.when(i == NBLK - 1)
    def _():
        z = pool_acc[...] / jnp.maximum(cnt_acc[...], 1.0)
        z = jnp.tanh(jnp.dot(z, w1_ref[...],
                             preferred_element_type=jnp.float32) + c1_ref[...])
        z = jnp.tanh(jnp.dot(z, w2_ref[...],
                             preferred_element_type=jnp.float32) + c2_ref[...])
        z = jnp.tanh(jnp.dot(z, w3_ref[...],
                             preferred_element_type=jnp.float32) + c3_ref[...])
        out_ref[...] = (jnp.dot(z, w4_ref[...],
                                preferred_element_type=jnp.float32)
                        + b4_ref[...])


def _final(agg, deg_carrier, r3, batch_col, w1, c1, w2, c2, w3, c3, w4, b4):
    full = lambda a: pl.BlockSpec(a.shape, lambda i: tuple(0 for _ in a.shape))
    return pl.pallas_call(
        _final_body,
        grid=(NBLK,),
        in_specs=[
            pl.BlockSpec((NC, BLK, TW), lambda i: (0, i, 0)),
            pl.BlockSpec((NC, BLK, TW), lambda i: (0, i, 0)),
            pl.BlockSpec((BLK, DH), lambda i: (i, 0)),
            pl.BlockSpec((BLK, 1), lambda i: (i, 0)),
            full(w1), full(c1), full(w2), full(c2),
            full(w3), full(c3), full(w4), full(b4),
        ],
        out_specs=pl.BlockSpec((G, NCLS), lambda i: (0, 0)),
        out_shape=jax.ShapeDtypeStruct((G, NCLS), jnp.float32),
        scratch_shapes=[
            pltpu.VMEM((G, DH), jnp.float32),
            pltpu.VMEM((G, 1), jnp.float32),
        ],
    )(agg, deg_carrier, r3, batch_col, w1, c1, w2, c2, w3, c3, w4, b4)


def kernel(x, edge_index, batch,
           cW1l, cW1r, cb1, cW2l, cW2r, cb2, cW3l, cW3r, cb3,
           W1, b1, g1, be1, rm1, rv1,
           W2, b2, g2, be2, rm2, rv2,
           W3, b3, g3, be3, rm3, rv3,
           W4, b4):
    src = edge_index[0]
    dst = edge_index[1]
    z = jnp.zeros((N, TW), jnp.float32)

    def fold(w, b, gm, be, rm, rv):
        s = gm * jax.lax.rsqrt(rv + 1e-5)
        return w * s[None, :], ((b - rm) * s + be).reshape(1, -1)

    w1f, c1 = fold(W1, b1, g1, be1, rm1, rv1)
    w2f, c2 = fold(W2, b2, g2, be2, rm2, rv2)
    w3f, c3 = fold(W3, b3, g3, be3, rm3, rv3)

    sc_pass = _make_sc_pass()
    p1, r1 = _dense_in(x, cW1l, cW1r, cb1)
    agg1 = sc_pass(src, dst, p1, z)
    p2, r2 = _dense_mid(agg1, agg1, r1, cW2l, cW2r, cb2)
    agg2 = sc_pass(src, dst, p2, z)
    p3, r3 = _dense_mid(agg2, agg1, r2, cW3l, cW3r, cb3)
    agg3 = sc_pass(src, dst, p3, z)
    return _final(agg3, agg1, r3, batch.reshape(N, 1),
                  w1f, c1, w2f, c2, w3f, c3, W4, b4.reshape(1, NCLS))
